# baseline (speedup 1.0000x reference)
import functools
import jax
import jax.numpy as jnp
from jax import lax
from jax.experimental import pallas as pl
from jax.experimental.pallas import tpu as pltpu
from jax.experimental.pallas import tpu_sc as plsc

NUM_SEGMENTS = 128
N = 100000
D = 256
DG = D // 16
AW = D + 16

NW = 32
CHUNK = 128
CH_PER_TILE = 12
RPW = CHUNK * CH_PER_TILE
SC_ROWS = NW * RPW
TC_ROWS = N - SC_ROWS

TC_BLK = 1024
TC_BLK0 = SC_ROWS // TC_BLK
TC_GRID = (N - SC_ROWS + TC_BLK - 1) // TC_BLK
WIN = 16


def _stage1_body(x_hbm, b_hbm, pacc_hbm, xbufs, idxbufs, acc_v, rsum,
                 sem0, sem1):
    cid = lax.axis_index("c")
    sid = lax.axis_index("s")
    wid = sid * 2 + cid
    lo = wid * RPW

    zeros16 = jnp.zeros((16,), jnp.float32)
    ones16 = jnp.ones((16,), jnp.float32)

    def _za(t, c):
        acc_v[t // (AW // 16), pl.ds((t % (AW // 16)) * 16, 16)] = zeros16
        return c
    lax.fori_loop(0, NUM_SEGMENTS * (AW // 16), _za, 0)

    for j in range(DG):
        rsum[j, :] = zeros16

    def _flush(cur, cnt):
        for j in range(DG):
            plsc.addupdate(acc_v.at[cur, pl.ds(j * 16, 16)], rsum[j, :])
            rsum[j, :] = zeros16
        plsc.addupdate(acc_v.at[cur, pl.ds(D, 16)],
                       jnp.full((16,), cnt, jnp.float32))

    sems = (sem0, sem1)

    def _issue(k, b):
        base = lo + k * CHUNK
        pltpu.async_copy(x_hbm.at[pl.ds(base, CHUNK)], xbufs.at[b], sems[b])
        pltpu.async_copy(b_hbm.at[pl.ds(base, CHUNK)], idxbufs.at[b], sems[b])

    def _wait(b):
        pltpu.make_async_copy(x_hbm.at[pl.ds(0, CHUNK)], xbufs.at[b],
                              sems[b]).wait()
        pltpu.make_async_copy(b_hbm.at[pl.ds(0, CHUNK)], idxbufs.at[b],
                              sems[b]).wait()

    def _accum_groups(ngroups, carry, bsel):
        xbuf = xbufs.at[bsel]
        idxbuf = idxbufs.at[bsel]

        def _group(g, carry):
            cur, cnt = carry
            ids = idxbuf[pl.ds(g * 16, 16)]
            first = ids[0]
            uniform = first == ids[15]

            def _fast(args):
                cur, cnt = args
                for j in range(DG):
                    a = xbuf[g * 16, pl.ds(j * 16, 16)]
                    for l in range(1, 16):
                        a = a + xbuf[g * 16 + l, pl.ds(j * 16, 16)]
                    rsum[j, :] = rsum[j, :] + a
                return (cur, cnt + 16.0)

            def _slow(args):
                cur, cnt = args

                @pl.when(cur >= 0)
                def _():
                    _flush(cur, cnt)

                for l in range(16):
                    b = ids[l]
                    for j in range(DG):
                        v = xbuf[g * 16 + l, pl.ds(j * 16, 16)]
                        plsc.addupdate(acc_v.at[b, pl.ds(j * 16, 16)], v)
                    plsc.addupdate(acc_v.at[b, pl.ds(D, 16)], ones16)
                return (ids[15], jnp.float32(0.0))

            return lax.cond(uniform & (first == cur), _fast, _slow, carry)
        return lax.fori_loop(0, ngroups, _group, carry)

    def _consume_chunk(carry, bsel):
        xbuf = xbufs.at[bsel]
        idxbuf = idxbufs.at[bsel]
        cur, cnt = carry
        first = idxbuf[pl.ds(0, 16)][0]
        last = idxbuf[pl.ds(CHUNK - 16, 16)][15]

        def _cfast(args):
            cur, cnt = args

            def _g(g, c):
                for j in range(DG):
                    a = xbuf[g * 16, pl.ds(j * 16, 16)]
                    for l in range(1, 16):
                        a = a + xbuf[g * 16 + l, pl.ds(j * 16, 16)]
                    rsum[j, :] = rsum[j, :] + a
                return c
            lax.fori_loop(0, CHUNK // 16, _g, 0)
            return (cur, cnt + float(CHUNK))

        def _cslow(args):
            return _accum_groups(CHUNK // 16, args, bsel)

        return lax.cond((first == last) & (first == cur),
                        _cfast, _cslow, carry)

    def _do(k, b, carry):
        @pl.when(k + 1 < CH_PER_TILE)
        def _():
            _issue(k + 1, 1 - b)
        _wait(b)
        return _consume_chunk(carry, b)

    _issue(0, 0)
    carry0 = (jnp.int32(-1), jnp.float32(0.0))

    def _pair(p, carry):
        k0 = 2 * p
        carry = _do(k0, 0, carry)
        carry = lax.cond(k0 + 1 < CH_PER_TILE,
                         lambda c: _do(k0 + 1, 1, c), lambda c: c, carry)
        return carry

    carry = lax.fori_loop(0, (CH_PER_TILE + 1) // 2, _pair, carry0)
    cur, cnt = carry

    @pl.when(cur >= 0)
    def _final_flush():
        _flush(cur, cnt)

    pltpu.sync_copy(acc_v, pacc_hbm.at[wid])


_stage1 = functools.partial(
    pl.kernel,
    mesh=plsc.VectorSubcoreMesh(core_axis_name="c", subcore_axis_name="s"),
    out_type=jax.ShapeDtypeStruct((NW, NUM_SEGMENTS, AW), jnp.float32),
    scratch_types=[
        pltpu.VMEM((2, CHUNK, D), jnp.float32),
        pltpu.VMEM((2, CHUNK), jnp.int32),
        pltpu.VMEM((NUM_SEGMENTS, AW), jnp.float32),
        pltpu.VMEM((DG, 16), jnp.float32),
        pltpu.SemaphoreType.DMA,
        pltpu.SemaphoreType.DMA,
    ],
)(_stage1_body)


def _tc_body(x_ref, b_ref, acc_ref):
    k = pl.program_id(0)

    @pl.when(k == 0)
    def _():
        acc_ref[...] = jnp.zeros_like(acc_ref)

    ids = b_ref[...]
    row = lax.iota(jnp.int32, TC_BLK) + (TC_BLK0 + k) * TC_BLK
    valid = row < N
    idv_min = jnp.min(jnp.where(valid, ids, NUM_SEGMENTS))
    idv_max = jnp.max(jnp.where(valid, ids, -1))

    def _window(_):
        wbase = pl.multiple_of(
            jnp.minimum((idv_min // 8) * 8, NUM_SEGMENTS - WIN), 8)
        wseg = lax.broadcasted_iota(jnp.int32, (WIN, TC_BLK), 0)
        oh = jnp.where(valid[None, :] & ((ids - wbase)[None, :] == wseg),
                       1.0, 0.0)
        acc_ref[pl.ds(wbase, WIN), :D] += jnp.dot(
            oh, x_ref[...], preferred_element_type=jnp.float32)
        acc_ref[pl.ds(wbase, WIN), D] += jnp.sum(oh, axis=1)
        return 0

    def _full(_):
        seg = lax.broadcasted_iota(jnp.int32, (NUM_SEGMENTS, TC_BLK), 0)
        oh = jnp.where(valid[None, :] & (ids[None, :] == seg), 1.0, 0.0)
        acc_ref[:, :D] += jnp.dot(oh, x_ref[...],
                                  preferred_element_type=jnp.float32)
        acc_ref[:, D] += jnp.sum(oh, axis=1)
        return 0

    lax.cond(idv_max - idv_min <= 8, _window, _full, 0)


_tc_part = pl.pallas_call(
    _tc_body,
    grid=(TC_GRID,),
    in_specs=[
        pl.BlockSpec((TC_BLK, D), lambda k: (TC_BLK0 + k, 0)),
        pl.BlockSpec((TC_BLK,), lambda k: (TC_BLK0 + k,)),
    ],
    out_specs=pl.BlockSpec((NUM_SEGMENTS, D + 1), lambda k: (0, 0)),
    out_shape=jax.ShapeDtypeStruct((NUM_SEGMENTS, D + 1), jnp.float32),
)


def _combine_body(pacc_ref, tacc_ref, o_ref):
    acc = jnp.sum(pacc_ref[...], axis=0)
    s = acc[:, :D] + tacc_ref[:, :D]
    c = acc[:, D] + tacc_ref[:, D]
    o_ref[...] = s / jnp.maximum(c, 1.0)[:, None]


_combine = pl.pallas_call(
    _combine_body,
    out_shape=jax.ShapeDtypeStruct((NUM_SEGMENTS, D), jnp.float32),
)


@jax.jit
def kernel(x, batch):
    batch = batch.astype(jnp.int32)
    tacc = _tc_part(x, batch)
    pacc = _stage1(x, batch)
    return _combine(pacc, tacc)


# speedup vs baseline: 1.0714x; 1.0714x over previous
import functools
import jax
import jax.numpy as jnp
from jax import lax
from jax.experimental import pallas as pl
from jax.experimental.pallas import tpu as pltpu
from jax.experimental.pallas import tpu_sc as plsc

NUM_SEGMENTS = 128
N = 100000
D = 256
DG = D // 16
AW = D + 16

NW = 32
CHUNK = 128
CH_PER_TILE = 10
RPW = CHUNK * CH_PER_TILE
SC_ROWS = NW * RPW
TC_ROWS = N - SC_ROWS

TC_BLK = 1024
TC_BLK0 = SC_ROWS // TC_BLK
TC_GRID = (N - SC_ROWS + TC_BLK - 1) // TC_BLK
WIN = 16


def _stage1_body(x_hbm, b_hbm, pacc_hbm, xbufs, idxbufs, acc_v, rsum,
                 sem0, sem1):
    cid = lax.axis_index("c")
    sid = lax.axis_index("s")
    wid = sid * 2 + cid
    lo = wid * RPW

    zeros16 = jnp.zeros((16,), jnp.float32)
    ones16 = jnp.ones((16,), jnp.float32)

    def _za(t, c):
        acc_v[t // (AW // 16), pl.ds((t % (AW // 16)) * 16, 16)] = zeros16
        return c
    lax.fori_loop(0, NUM_SEGMENTS * (AW // 16), _za, 0)

    for j in range(DG):
        rsum[j, :] = zeros16

    def _flush(cur, cnt):
        for j in range(DG):
            plsc.addupdate(acc_v.at[cur, pl.ds(j * 16, 16)], rsum[j, :])
            rsum[j, :] = zeros16
        plsc.addupdate(acc_v.at[cur, pl.ds(D, 16)],
                       jnp.full((16,), cnt, jnp.float32))

    sems = (sem0, sem1)

    def _issue(k, b):
        base = lo + k * CHUNK
        pltpu.async_copy(x_hbm.at[pl.ds(base, CHUNK)], xbufs.at[b], sems[b])
        pltpu.async_copy(b_hbm.at[pl.ds(base, CHUNK)], idxbufs.at[b], sems[b])

    def _wait(b):
        pltpu.make_async_copy(x_hbm.at[pl.ds(0, CHUNK)], xbufs.at[b],
                              sems[b]).wait()
        pltpu.make_async_copy(b_hbm.at[pl.ds(0, CHUNK)], idxbufs.at[b],
                              sems[b]).wait()

    def _accum_groups(ngroups, carry, bsel):
        xbuf = xbufs.at[bsel]
        idxbuf = idxbufs.at[bsel]

        def _group(g, carry):
            cur, cnt = carry
            ids = idxbuf[pl.ds(g * 16, 16)]
            first = ids[0]
            uniform = first == ids[15]

            def _fast(args):
                cur, cnt = args
                for j in range(DG):
                    a = xbuf[g * 16, pl.ds(j * 16, 16)]
                    for l in range(1, 16):
                        a = a + xbuf[g * 16 + l, pl.ds(j * 16, 16)]
                    rsum[j, :] = rsum[j, :] + a
                return (cur, cnt + 16.0)

            def _slow(args):
                cur, cnt = args

                @pl.when(cur >= 0)
                def _():
                    _flush(cur, cnt)

                for l in range(16):
                    b = ids[l]
                    for j in range(DG):
                        v = xbuf[g * 16 + l, pl.ds(j * 16, 16)]
                        plsc.addupdate(acc_v.at[b, pl.ds(j * 16, 16)], v)
                    plsc.addupdate(acc_v.at[b, pl.ds(D, 16)], ones16)
                return (ids[15], jnp.float32(0.0))

            return lax.cond(uniform & (first == cur), _fast, _slow, carry)
        return lax.fori_loop(0, ngroups, _group, carry)

    def _consume_chunk(carry, bsel):
        xbuf = xbufs.at[bsel]
        idxbuf = idxbufs.at[bsel]
        cur, cnt = carry
        first = idxbuf[pl.ds(0, 16)][0]
        last = idxbuf[pl.ds(CHUNK - 16, 16)][15]

        def _cfast(args):
            cur, cnt = args

            def _g(g, c):
                for j in range(DG):
                    a = xbuf[g * 16, pl.ds(j * 16, 16)]
                    for l in range(1, 16):
                        a = a + xbuf[g * 16 + l, pl.ds(j * 16, 16)]
                    rsum[j, :] = rsum[j, :] + a
                return c
            lax.fori_loop(0, CHUNK // 16, _g, 0)
            return (cur, cnt + float(CHUNK))

        def _cslow(args):
            return _accum_groups(CHUNK // 16, args, bsel)

        return lax.cond((first == last) & (first == cur),
                        _cfast, _cslow, carry)

    def _do(k, b, carry):
        @pl.when(k + 1 < CH_PER_TILE)
        def _():
            _issue(k + 1, 1 - b)
        _wait(b)
        return _consume_chunk(carry, b)

    _issue(0, 0)
    carry0 = (jnp.int32(-1), jnp.float32(0.0))

    def _pair(p, carry):
        k0 = 2 * p
        carry = _do(k0, 0, carry)
        carry = lax.cond(k0 + 1 < CH_PER_TILE,
                         lambda c: _do(k0 + 1, 1, c), lambda c: c, carry)
        return carry

    carry = lax.fori_loop(0, (CH_PER_TILE + 1) // 2, _pair, carry0)
    cur, cnt = carry

    @pl.when(cur >= 0)
    def _final_flush():
        _flush(cur, cnt)

    pltpu.sync_copy(acc_v, pacc_hbm.at[wid])


_stage1 = functools.partial(
    pl.kernel,
    mesh=plsc.VectorSubcoreMesh(core_axis_name="c", subcore_axis_name="s"),
    out_type=jax.ShapeDtypeStruct((NW, NUM_SEGMENTS, AW), jnp.float32),
    scratch_types=[
        pltpu.VMEM((2, CHUNK, D), jnp.float32),
        pltpu.VMEM((2, CHUNK), jnp.int32),
        pltpu.VMEM((NUM_SEGMENTS, AW), jnp.float32),
        pltpu.VMEM((DG, 16), jnp.float32),
        pltpu.SemaphoreType.DMA,
        pltpu.SemaphoreType.DMA,
    ],
)(_stage1_body)


def _tc_body(x_ref, b_ref, acc_ref):
    k = pl.program_id(0)

    @pl.when(k == 0)
    def _():
        acc_ref[...] = jnp.zeros_like(acc_ref)

    ids = b_ref[...]
    row = lax.iota(jnp.int32, TC_BLK) + (TC_BLK0 + k) * TC_BLK
    valid = row < N
    idv_min = jnp.min(jnp.where(valid, ids, NUM_SEGMENTS))
    idv_max = jnp.max(jnp.where(valid, ids, -1))

    def _window(_):
        wbase = pl.multiple_of(
            jnp.minimum((idv_min // 8) * 8, NUM_SEGMENTS - WIN), 8)
        wseg = lax.broadcasted_iota(jnp.int32, (WIN, TC_BLK), 0)
        oh = jnp.where(valid[None, :] & ((ids - wbase)[None, :] == wseg),
                       1.0, 0.0)
        acc_ref[pl.ds(wbase, WIN), :D] += jnp.dot(
            oh, x_ref[...], preferred_element_type=jnp.float32)
        acc_ref[pl.ds(wbase, WIN), D] += jnp.sum(oh, axis=1)
        return 0

    def _full(_):
        seg = lax.broadcasted_iota(jnp.int32, (NUM_SEGMENTS, TC_BLK), 0)
        oh = jnp.where(valid[None, :] & (ids[None, :] == seg), 1.0, 0.0)
        acc_ref[:, :D] += jnp.dot(oh, x_ref[...],
                                  preferred_element_type=jnp.float32)
        acc_ref[:, D] += jnp.sum(oh, axis=1)
        return 0

    lax.cond(idv_max - idv_min <= 8, _window, _full, 0)


_tc_part = pl.pallas_call(
    _tc_body,
    grid=(TC_GRID,),
    in_specs=[
        pl.BlockSpec((TC_BLK, D), lambda k: (TC_BLK0 + k, 0)),
        pl.BlockSpec((TC_BLK,), lambda k: (TC_BLK0 + k,)),
    ],
    out_specs=pl.BlockSpec((NUM_SEGMENTS, D + 1), lambda k: (0, 0)),
    out_shape=jax.ShapeDtypeStruct((NUM_SEGMENTS, D + 1), jnp.float32),
)


def _combine_body(pacc_ref, tacc_ref, o_ref):
    acc = jnp.sum(pacc_ref[...], axis=0)
    s = acc[:, :D] + tacc_ref[:, :D]
    c = acc[:, D] + tacc_ref[:, D]
    o_ref[...] = s / jnp.maximum(c, 1.0)[:, None]


_combine = pl.pallas_call(
    _combine_body,
    out_shape=jax.ShapeDtypeStruct((NUM_SEGMENTS, D), jnp.float32),
)


@jax.jit
def kernel(x, batch):
    batch = batch.astype(jnp.int32)
    tacc = _tc_part(x, batch)
    pacc = _stage1(x, batch)
    return _combine(pacc, tacc)


# speedup vs baseline: 1.1064x; 1.0327x over previous
import functools
import jax
import jax.numpy as jnp
from jax import lax
from jax.experimental import pallas as pl
from jax.experimental.pallas import tpu as pltpu
from jax.experimental.pallas import tpu_sc as plsc

NUM_SEGMENTS = 128
N = 100000
D = 256
DG = D // 16
AW = D + 16

NW = 32
CHUNK = 128
CH_PER_TILE = 9
RPW = CHUNK * CH_PER_TILE
SC_ROWS = NW * RPW
TC_ROWS = N - SC_ROWS

TC_BLK = 2048
TC_BLK0 = SC_ROWS // TC_BLK
TC_GRID = (N - SC_ROWS + TC_BLK - 1) // TC_BLK
WIN = 16


def _stage1_body(x_hbm, b_hbm, pacc_hbm, xbufs, idxbufs, acc_v, rsum,
                 sem0, sem1):
    cid = lax.axis_index("c")
    sid = lax.axis_index("s")
    wid = sid * 2 + cid
    lo = wid * RPW

    zeros16 = jnp.zeros((16,), jnp.float32)
    ones16 = jnp.ones((16,), jnp.float32)

    def _za(t, c):
        acc_v[t // (AW // 16), pl.ds((t % (AW // 16)) * 16, 16)] = zeros16
        return c
    lax.fori_loop(0, NUM_SEGMENTS * (AW // 16), _za, 0)

    for j in range(DG):
        rsum[j, :] = zeros16

    def _flush(cur, cnt):
        for j in range(DG):
            plsc.addupdate(acc_v.at[cur, pl.ds(j * 16, 16)], rsum[j, :])
            rsum[j, :] = zeros16
        plsc.addupdate(acc_v.at[cur, pl.ds(D, 16)],
                       jnp.full((16,), cnt, jnp.float32))

    sems = (sem0, sem1)

    def _issue(k, b):
        base = lo + k * CHUNK
        pltpu.async_copy(x_hbm.at[pl.ds(base, CHUNK)], xbufs.at[b], sems[b])
        pltpu.async_copy(b_hbm.at[pl.ds(base, CHUNK)], idxbufs.at[b], sems[b])

    def _wait(b):
        pltpu.make_async_copy(x_hbm.at[pl.ds(0, CHUNK)], xbufs.at[b],
                              sems[b]).wait()
        pltpu.make_async_copy(b_hbm.at[pl.ds(0, CHUNK)], idxbufs.at[b],
                              sems[b]).wait()

    def _accum_groups(ngroups, carry, bsel):
        xbuf = xbufs.at[bsel]
        idxbuf = idxbufs.at[bsel]

        def _group(g, carry):
            cur, cnt = carry
            ids = idxbuf[pl.ds(g * 16, 16)]
            first = ids[0]
            uniform = first == ids[15]

            def _fast(args):
                cur, cnt = args
                for j in range(DG):
                    a = xbuf[g * 16, pl.ds(j * 16, 16)]
                    for l in range(1, 16):
                        a = a + xbuf[g * 16 + l, pl.ds(j * 16, 16)]
                    rsum[j, :] = rsum[j, :] + a
                return (cur, cnt + 16.0)

            def _slow(args):
                cur, cnt = args

                @pl.when(cur >= 0)
                def _():
                    _flush(cur, cnt)

                for l in range(16):
                    b = ids[l]
                    for j in range(DG):
                        v = xbuf[g * 16 + l, pl.ds(j * 16, 16)]
                        plsc.addupdate(acc_v.at[b, pl.ds(j * 16, 16)], v)
                    plsc.addupdate(acc_v.at[b, pl.ds(D, 16)], ones16)
                return (ids[15], jnp.float32(0.0))

            return lax.cond(uniform & (first == cur), _fast, _slow, carry)
        return lax.fori_loop(0, ngroups, _group, carry)

    def _consume_chunk(carry, bsel):
        xbuf = xbufs.at[bsel]
        idxbuf = idxbufs.at[bsel]
        cur, cnt = carry
        first = idxbuf[pl.ds(0, 16)][0]
        last = idxbuf[pl.ds(CHUNK - 16, 16)][15]

        def _cfast(args):
            cur, cnt = args

            def _g(g, c):
                for j in range(DG):
                    a = xbuf[g * 16, pl.ds(j * 16, 16)]
                    for l in range(1, 16):
                        a = a + xbuf[g * 16 + l, pl.ds(j * 16, 16)]
                    rsum[j, :] = rsum[j, :] + a
                return c
            lax.fori_loop(0, CHUNK // 16, _g, 0)
            return (cur, cnt + float(CHUNK))

        def _cslow(args):
            return _accum_groups(CHUNK // 16, args, bsel)

        return lax.cond((first == last) & (first == cur),
                        _cfast, _cslow, carry)

    def _do(k, b, carry):
        @pl.when(k + 1 < CH_PER_TILE)
        def _():
            _issue(k + 1, 1 - b)
        _wait(b)
        return _consume_chunk(carry, b)

    _issue(0, 0)
    carry0 = (jnp.int32(-1), jnp.float32(0.0))

    def _pair(p, carry):
        k0 = 2 * p
        carry = _do(k0, 0, carry)
        carry = lax.cond(k0 + 1 < CH_PER_TILE,
                         lambda c: _do(k0 + 1, 1, c), lambda c: c, carry)
        return carry

    carry = lax.fori_loop(0, (CH_PER_TILE + 1) // 2, _pair, carry0)
    cur, cnt = carry

    @pl.when(cur >= 0)
    def _final_flush():
        _flush(cur, cnt)

    pltpu.sync_copy(acc_v, pacc_hbm.at[wid])


_stage1 = functools.partial(
    pl.kernel,
    mesh=plsc.VectorSubcoreMesh(core_axis_name="c", subcore_axis_name="s"),
    out_type=jax.ShapeDtypeStruct((NW, NUM_SEGMENTS, AW), jnp.float32),
    scratch_types=[
        pltpu.VMEM((2, CHUNK, D), jnp.float32),
        pltpu.VMEM((2, CHUNK), jnp.int32),
        pltpu.VMEM((NUM_SEGMENTS, AW), jnp.float32),
        pltpu.VMEM((DG, 16), jnp.float32),
        pltpu.SemaphoreType.DMA,
        pltpu.SemaphoreType.DMA,
    ],
)(_stage1_body)


def _tc_body(x_ref, b_ref, acc_ref):
    k = pl.program_id(0)

    @pl.when(k == 0)
    def _():
        acc_ref[...] = jnp.zeros_like(acc_ref)

    ids = b_ref[...]
    row = lax.iota(jnp.int32, TC_BLK) + (TC_BLK0 + k) * TC_BLK
    valid = row < N
    idv_min = jnp.min(jnp.where(valid, ids, NUM_SEGMENTS))
    idv_max = jnp.max(jnp.where(valid, ids, -1))

    def _window(_):
        wbase = pl.multiple_of(
            jnp.minimum((idv_min // 8) * 8, NUM_SEGMENTS - WIN), 8)
        wseg = lax.broadcasted_iota(jnp.int32, (WIN, TC_BLK), 0)
        oh = jnp.where(valid[None, :] & ((ids - wbase)[None, :] == wseg),
                       1.0, 0.0)
        acc_ref[pl.ds(wbase, WIN), :D] += jnp.dot(
            oh, x_ref[...], preferred_element_type=jnp.float32)
        acc_ref[pl.ds(wbase, WIN), D] += jnp.sum(oh, axis=1)
        return 0

    def _full(_):
        seg = lax.broadcasted_iota(jnp.int32, (NUM_SEGMENTS, TC_BLK), 0)
        oh = jnp.where(valid[None, :] & (ids[None, :] == seg), 1.0, 0.0)
        acc_ref[:, :D] += jnp.dot(oh, x_ref[...],
                                  preferred_element_type=jnp.float32)
        acc_ref[:, D] += jnp.sum(oh, axis=1)
        return 0

    lax.cond(idv_max - idv_min <= 8, _window, _full, 0)


_tc_part = pl.pallas_call(
    _tc_body,
    grid=(TC_GRID,),
    in_specs=[
        pl.BlockSpec((TC_BLK, D), lambda k: (TC_BLK0 + k, 0)),
        pl.BlockSpec((TC_BLK,), lambda k: (TC_BLK0 + k,)),
    ],
    out_specs=pl.BlockSpec((NUM_SEGMENTS, D + 1), lambda k: (0, 0)),
    out_shape=jax.ShapeDtypeStruct((NUM_SEGMENTS, D + 1), jnp.float32),
)


def _combine_body(pacc_ref, tacc_ref, o_ref):
    acc = jnp.sum(pacc_ref[...], axis=0)
    s = acc[:, :D] + tacc_ref[:, :D]
    c = acc[:, D] + tacc_ref[:, D]
    o_ref[...] = s / jnp.maximum(c, 1.0)[:, None]


_combine = pl.pallas_call(
    _combine_body,
    out_shape=jax.ShapeDtypeStruct((NUM_SEGMENTS, D), jnp.float32),
)


@jax.jit
def kernel(x, batch):
    batch = batch.astype(jnp.int32)
    tacc = _tc_part(x, batch)
    pacc = _stage1(x, batch)
    return _combine(pacc, tacc)
